# baseline (speedup 1.0000x reference)
import functools

import jax
import jax.numpy as jnp
from jax import lax
from jax.experimental import pallas as pl
from jax.experimental.pallas import tpu as pltpu
from jax.experimental.pallas import tpu_sc as plsc

NC = 2
NS = 16
L = 16
NW = NC * NS

CE = 4


@functools.lru_cache(maxsize=None)
def _build(B, S, D, V):
    assert B % NW == 0 and D % L == 0
    bpw = B // NW
    nchunks = bpw // CE
    assert bpw % CE == 0 and nchunks % 2 == 0
    inv_s = 1.0 / S

    mesh = plsc.VectorSubcoreMesh(
        core_axis_name="c", subcore_axis_name="s", num_cores=NC,
        num_subcores=NS)

    @functools.partial(
        pl.kernel,
        out_type=jax.ShapeDtypeStruct((B, D), jnp.float32),
        mesh=mesh,
        scratch_types=[
            pltpu.VMEM((bpw * S,), jnp.int32),
            pltpu.VMEM((CE * S, D), jnp.float32),
            pltpu.VMEM((CE * S, D), jnp.float32),
            pltpu.VMEM((CE, D), jnp.float32),
            pltpu.VMEM((CE, D), jnp.float32),
            pltpu.SemaphoreType.DMA,
            pltpu.SemaphoreType.DMA,
            pltpu.SemaphoreType.DMA,
            pltpu.SemaphoreType.DMA,
        ],
    )
    def emb_kernel(texts_h, emb_h, out_h, idx_v, rows0, rows1, outb0, outb1,
                   gsem0, gsem1, osem0, osem1):
        w = lax.axis_index("s") * NC + lax.axis_index("c")
        base = w * bpw
        pltpu.sync_copy(texts_h.at[pl.ds(base * S, bpw * S)], idx_v)

        rows = (rows0, rows1)
        outb = (outb0, outb1)
        gsem = (gsem0, gsem1)
        osem = (osem0, osem1)

        def gather(g, buf):
            pltpu.async_copy(
                emb_h.at[idx_v.at[pl.ds(g * (CE * S), CE * S)]],
                rows[buf], gsem[buf])

        def reduce_chunk(g, buf):
            rows_v, outb_v = rows[buf], outb[buf]

            def elem(e2, carry):
                for u in range(2):
                    e = 2 * e2 + u
                    r0 = e * S
                    for c in range(D // L):
                        ds = pl.ds(c * L, L)
                        vals = [rows_v[r0 + r, ds] for r in range(S)]
                        while len(vals) > 1:
                            vals = [vals[i] + vals[i + 1]
                                    for i in range(0, len(vals), 2)]
                        outb_v[e, ds] = vals[0] * inv_s
                return carry

            lax.fori_loop(0, CE // 2, elem, 0)
            pltpu.async_copy(outb_v, out_h.at[pl.ds(base + g * CE, CE)],
                             osem[buf])

        gather(0, 0)
        gather(1, 1)

        def pair_body(g2, carry):
            g = 2 * g2
            for buf in range(2):
                gc = g + buf
                pltpu.make_async_copy(
                    emb_h.at[idx_v.at[pl.ds(0, CE * S)]], rows[buf],
                    gsem[buf]).wait()

                @pl.when(g2 > 0)
                def _():
                    pltpu.make_async_copy(
                        outb[buf], out_h.at[pl.ds(base, CE)],
                        osem[buf]).wait()

                reduce_chunk(gc, buf)

                @pl.when(gc + 2 < nchunks)
                def _():
                    gather(gc + 2, buf)
            return carry

        lax.fori_loop(0, nchunks // 2, pair_body, 0)
        for buf in range(2):
            pltpu.make_async_copy(
                outb[buf], out_h.at[pl.ds(base, CE)], osem[buf]).wait()

    return emb_kernel


def kernel(texts, emb):
    B, S = texts.shape
    V, D = emb.shape
    texts_flat = texts.reshape(-1).astype(jnp.int32)
    return _build(B, S, D, V)(texts_flat, emb)


# speedup vs baseline: 1.5629x; 1.5629x over previous
import functools

import jax
import jax.numpy as jnp
from jax import lax
from jax.experimental import pallas as pl
from jax.experimental.pallas import tpu as pltpu
from jax.experimental.pallas import tpu_sc as plsc

NC = 2
NS = 16
L = 16
NW = NC * NS

CE = 4


@functools.lru_cache(maxsize=None)
def _build(B, S, D, V):
    assert B % NW == 0 and D % L == 0
    bpw = B // NW
    nchunks = bpw // CE
    assert bpw % CE == 0 and nchunks % 2 == 0
    inv_s = 1.0 / S

    mesh = plsc.VectorSubcoreMesh(
        core_axis_name="c", subcore_axis_name="s", num_cores=NC,
        num_subcores=NS)

    @functools.partial(
        pl.kernel,
        out_type=jax.ShapeDtypeStruct((B, D), jnp.float32),
        mesh=mesh,
        scratch_types=[
            pltpu.VMEM((bpw * S,), jnp.int32),
            pltpu.VMEM((CE * S, D), jnp.float32),
            pltpu.VMEM((CE * S, D), jnp.float32),
            pltpu.VMEM((CE, D), jnp.float32),
            pltpu.VMEM((CE, D), jnp.float32),
            pltpu.SemaphoreType.DMA,
            pltpu.SemaphoreType.DMA,
            pltpu.SemaphoreType.DMA,
            pltpu.SemaphoreType.DMA,
        ],
    )
    def emb_kernel(texts_h, emb_h, out_h, idx_v, rows0, rows1, outb0, outb1,
                   gsem0, gsem1, osem0, osem1):
        w = lax.axis_index("s") * NC + lax.axis_index("c")
        base = w * bpw
        pltpu.sync_copy(texts_h.at[pl.ds(base * S, bpw * S)], idx_v)

        rows = (rows0, rows1)
        outb = (outb0, outb1)
        gsem = (gsem0, gsem1)
        osem = (osem0, osem1)

        def gather(g, buf):
            pltpu.async_copy(
                emb_h.at[idx_v.at[pl.ds(g * (CE * S), CE * S)]],
                rows[buf], gsem[buf])

        def reduce_chunk(g, buf):
            rows_v, outb_v = rows[buf], outb[buf]

            def elem(e, carry):
                r0 = e * S
                for c in range(D // L):
                    ds = pl.ds(c * L, L)
                    vals = [rows_v[r0 + r, ds] for r in range(S)]
                    while len(vals) > 1:
                        vals = [vals[i] + vals[i + 1]
                                for i in range(0, len(vals), 2)]
                    outb_v[e, ds] = vals[0] * inv_s
                return carry

            lax.fori_loop(0, CE, elem, 0)
            pltpu.async_copy(outb_v, out_h.at[pl.ds(base + g * CE, CE)],
                             osem[buf])

        gather(0, 0)
        gather(1, 1)

        def pair_body(g2, carry):
            g = 2 * g2
            for buf in range(2):
                gc = g + buf
                pltpu.make_async_copy(
                    emb_h.at[idx_v.at[pl.ds(0, CE * S)]], rows[buf],
                    gsem[buf]).wait()

                @pl.when(g2 > 0)
                def _():
                    pltpu.make_async_copy(
                        outb[buf], out_h.at[pl.ds(base, CE)],
                        osem[buf]).wait()

                reduce_chunk(gc, buf)

                @pl.when(gc + 2 < nchunks)
                def _():
                    gather(gc + 2, buf)
            return carry

        lax.fori_loop(0, nchunks // 2, pair_body, 0)
        for buf in range(2):
            pltpu.make_async_copy(
                outb[buf], out_h.at[pl.ds(base, CE)], osem[buf]).wait()

    return emb_kernel


def kernel(texts, emb):
    B, S = texts.shape
    V, D = emb.shape
    texts_flat = texts.reshape(-1).astype(jnp.int32)
    return _build(B, S, D, V)(texts_flat, emb)
